# baseline (speedup 1.0000x reference)
import functools

import jax
import jax.numpy as jnp
from jax import lax
from jax.experimental import pallas as pl
from jax.experimental.pallas import tpu as pltpu
from jax.experimental.pallas import tpu_sc as plsc

NUM_WORKERS = 32
BLK = 4096
Q = BLK // 4
GROUPS = 6400
G_PER_W = GROUPS // NUM_WORKERS


def _repack_table(table_t):
    v = table_t.shape[1]
    nblk = (v + BLK - 1) // BLK

    def body(t_ref, o_ref):
        x = t_ref[...].T
        o_ref[...] = jnp.concatenate(
            [x[a * Q:(a + 1) * Q, :] for a in range(4)], axis=1)

    return pl.pallas_call(
        body,
        grid=(nblk,),
        in_specs=[pl.BlockSpec((32, BLK), lambda i: (0, i))],
        out_specs=pl.BlockSpec((Q, 128), lambda i: (i, 0)),
        out_shape=jax.ShapeDtypeStruct((nblk * Q, 128), jnp.float32),
    )(table_t)


def _sc_gather(idx2, tab32):
    mesh = plsc.VectorSubcoreMesh(core_axis_name="c", subcore_axis_name="s")

    @functools.partial(
        pl.kernel,
        out_type=jax.ShapeDtypeStruct((204800, 128), jnp.float32),
        mesh=mesh,
        scratch_types=[
            pltpu.VMEM((G_PER_W, 128), jnp.int32),
            pltpu.VMEM((4, 128, 32), jnp.float32),
            pltpu.VMEM((2, 32, 128), jnp.float32),
            pltpu.SemaphoreType.DMA,
            pltpu.SemaphoreType.DMA,
        ],
        compiler_params=pltpu.CompilerParams(
            use_tc_tiling_on_sc=False, needs_layout_passes=False),
    )
    def body(idx_hbm, tab_hbm, out_hbm, ridx_v, rows_v, f_v, gsem, ssem):
        wid = lax.axis_index("s") * 2 + lax.axis_index("c")
        g0 = wid * G_PER_W
        pltpu.sync_copy(idx_hbm.at[pl.ds(g0, G_PER_W)], ridx_v)

        def xform(i, carry):
            for k in range(8):
                t = ridx_v[i, pl.ds(k * 16, 16)]
                row = ((t & ~(BLK - 1))
                       + ((t & (Q - 1)) << 2)
                       + ((t >> 10) & 3))
                ridx_v[i, pl.ds(k * 16, 16)] = row
            return carry

        lax.fori_loop(0, G_PER_W, xform, 0)

        iota16 = lax.iota(jnp.int32, 16)
        rowvs = [iota16 + (b8 * 16) for b8 in range(8)]

        def gstart(j, slot):
            pltpu.async_copy(tab_hbm.at[ridx_v.at[j]], rows_v.at[slot], gsem)

        for jp in range(3):
            gstart(jp, jp)

        def do_group(j, slot, fslot):
            @pl.when(j + 3 < G_PER_W)
            def _():
                gstart(j + 3, (slot + 3) % 4)

            pltpu.make_async_copy(
                tab_hbm.at[ridx_v.at[j]], rows_v.at[slot], gsem).wait()

            @pl.when(j >= 2)
            def _():
                pltpu.make_async_copy(
                    f_v.at[0], out_hbm.at[pl.ds(0, 32)], ssem).wait()

            rv = rows_v.at[slot]
            fv = f_v.at[fslot]
            for f0 in range(32):
                colv = (iota16 + f0) & 31
                vals = [plsc.load_gather(rv, [rowvs[b8], colv])
                        for b8 in range(8)]
                for b8 in range(8):
                    plsc.store_scatter(fv, [colv, rowvs[b8]], vals[b8])

            g = g0 + j
            lt = g // 256
            rem = g - lt * 256
            tc = rem // 8
            rr = rem - tc * 8
            base = (lt * 8 + rr) * 1024 + tc * 8
            for tr in range(4):
                pltpu.async_copy(
                    fv.at[pl.ds(tr * 8, 8)],
                    out_hbm.at[pl.ds(base + tr * 256, 8)], ssem)

        def step4(jj, carry):
            for p in range(4):
                do_group(jj * 4 + p, p, p % 2)
            return carry

        lax.fori_loop(0, G_PER_W // 4, step4, 0)

        for _u in range(2):
            pltpu.make_async_copy(
                f_v.at[0], out_hbm.at[pl.ds(0, 32)], ssem).wait()

    return body(idx2, tab32)


def kernel(tokens, table):
    table_t = table.T
    table_lin = _repack_table(table_t)
    tab32 = table_lin.reshape(-1, 32)
    tok = tokens.astype(jnp.int32)
    tok_n = (tok.T.reshape(25, 8, 32, 128)
             .transpose(0, 2, 1, 3)
             .reshape(GROUPS, 128))
    out = _sc_gather(tok_n, tab32)
    return (out.reshape(200, 4, 32, 8, 128)
            .transpose(2, 4, 0, 1, 3)
            .reshape(4096, 200, 32))


# speedup vs baseline: 1.0311x; 1.0311x over previous
import functools

import jax
import jax.numpy as jnp
from jax import lax
from jax.experimental import pallas as pl
from jax.experimental.pallas import tpu as pltpu
from jax.experimental.pallas import tpu_sc as plsc

NUM_WORKERS = 32
BLK = 4096
Q = BLK // 4
GROUPS = 6400
G_PER_W = GROUPS // NUM_WORKERS


def _repack_table(table_t):
    v = table_t.shape[1]
    nblk = (v + BLK - 1) // BLK

    def body(t_ref, o_ref):
        x = t_ref[...].T
        o_ref[...] = jnp.concatenate(
            [x[a * Q:(a + 1) * Q, :] for a in range(4)], axis=1)

    return pl.pallas_call(
        body,
        grid=(nblk,),
        in_specs=[pl.BlockSpec((32, BLK), lambda i: (0, i))],
        out_specs=pl.BlockSpec((Q, 128), lambda i: (i, 0)),
        out_shape=jax.ShapeDtypeStruct((nblk * Q, 128), jnp.float32),
    )(table_t)


def _sc_gather(idx2, tab32):
    mesh = plsc.VectorSubcoreMesh(core_axis_name="c", subcore_axis_name="s")

    @functools.partial(
        pl.kernel,
        out_type=jax.ShapeDtypeStruct((204800, 128), jnp.float32),
        mesh=mesh,
        scratch_types=[
            pltpu.VMEM((G_PER_W, 128), jnp.int32),
            pltpu.VMEM((2, 128, 32), jnp.float32),
            pltpu.VMEM((2, 32, 128), jnp.float32),
            pltpu.SemaphoreType.DMA,
            pltpu.SemaphoreType.DMA,
        ],
        compiler_params=pltpu.CompilerParams(
            use_tc_tiling_on_sc=False, needs_layout_passes=False),
    )
    def body(idx_hbm, tab_hbm, out_hbm, ridx_v, rows_v, f_v, gsem, ssem):
        wid = lax.axis_index("s") * 2 + lax.axis_index("c")
        g0 = wid * G_PER_W
        pltpu.sync_copy(idx_hbm.at[pl.ds(g0, G_PER_W)], ridx_v)

        def xform(i, carry):
            for k in range(8):
                t = ridx_v[i, pl.ds(k * 16, 16)]
                row = ((t & ~(BLK - 1))
                       + ((t & (Q - 1)) << 2)
                       + ((t >> 10) & 3))
                ridx_v[i, pl.ds(k * 16, 16)] = row
            return carry

        lax.fori_loop(0, G_PER_W, xform, 0)

        iota16 = lax.iota(jnp.int32, 16)
        rowvs = [iota16 + (b8 * 16) for b8 in range(8)]

        def gstart(j, slot):
            pltpu.async_copy(tab_hbm.at[ridx_v.at[j]], rows_v.at[slot], gsem)

        gstart(0, 0)

        def do_group(j, slot, fslot):
            @pl.when(j + 1 < G_PER_W)
            def _():
                gstart(j + 1, 1 - slot)

            pltpu.make_async_copy(
                tab_hbm.at[ridx_v.at[j]], rows_v.at[slot], gsem).wait()

            @pl.when(j >= 2)
            def _():
                pltpu.make_async_copy(
                    f_v.at[0], out_hbm.at[pl.ds(0, 32)], ssem).wait()

            rv = rows_v.at[slot]
            fv = f_v.at[fslot]
            for f0 in range(32):
                colv = (iota16 + f0) & 31
                vals = [plsc.load_gather(rv, [rowvs[b8], colv])
                        for b8 in range(8)]
                for b8 in range(8):
                    plsc.store_scatter(fv, [colv, rowvs[b8]], vals[b8])

            g = g0 + j
            lt = g // 256
            rem = g - lt * 256
            tc = rem // 8
            rr = rem - tc * 8
            base = (lt * 8 + rr) * 1024 + tc * 8
            for tr in range(4):
                pltpu.async_copy(
                    fv.at[pl.ds(tr * 8, 8)],
                    out_hbm.at[pl.ds(base + tr * 256, 8)], ssem)

        def step2(jj, carry):
            for p in range(2):
                do_group(jj * 2 + p, p, p)
            return carry

        lax.fori_loop(0, G_PER_W // 2, step2, 0)

        for _u in range(2):
            pltpu.make_async_copy(
                f_v.at[0], out_hbm.at[pl.ds(0, 32)], ssem).wait()

    return body(idx2, tab32)


def kernel(tokens, table):
    table_t = table.T
    table_lin = _repack_table(table_t)
    tab32 = table_lin.reshape(-1, 32)
    tok = tokens.astype(jnp.int32)
    tok_n = (tok.T.reshape(25, 8, 32, 128)
             .transpose(0, 2, 1, 3)
             .reshape(GROUPS, 128))
    out = _sc_gather(tok_n, tab32)
    return (out.reshape(200, 4, 32, 8, 128)
            .transpose(2, 4, 0, 1, 3)
            .reshape(4096, 200, 32))
